# baseline (speedup 1.0000x reference)
import functools

import jax
import jax.numpy as jnp
from jax import lax
from jax.experimental import pallas as pl
from jax.experimental.pallas import tpu as pltpu
from jax.experimental.pallas import tpu_sc as plsc

NN = 10000
NP = 10112
D_IN = 128
D_HID = 128
D_OUT = 64
EDGES = 320000
CH = 128
EP = 323584
NCH = EP // CH
NC = 2
NS = 16
TCH = NCH // (NC * NS)
RPT = NP // NS

_mesh = lambda: plsc.VectorSubcoreMesh(
    core_axis_name="c", subcore_axis_name="s", num_cores=NC, num_subcores=NS)


def _zero_vmem_rows(ref, nrows, width):
  def zrow(i, carry):
    for k in range(width // 16):
      ref[i, pl.ds(k * 16, 16)] = jnp.zeros((16,), jnp.float32)
    return carry
  lax.fori_loop(0, nrows, zrow, None)


def _zero_acc_slice(zbuf, acc, base):
  for t in range(RPT // CH):
    pltpu.sync_copy(zbuf, acc.at[pl.ds(base + t * CH, CH)])
  rem = RPT % CH
  if rem:
    pltpu.sync_copy(zbuf.at[pl.ds(0, rem)], acc.at[pl.ds(base + (RPT // CH) * CH, rem)])


def _make_deg_kernel():
  @functools.partial(
      pl.kernel,
      out_type=jax.ShapeDtypeStruct((NC, NP, D_HID), jnp.float32),
      mesh=_mesh(),
      scratch_types=[
          pltpu.VMEM((TCH, CH), jnp.int32),
          pltpu.VMEM((CH, D_HID), jnp.float32),
          pltpu.VMEM((CH, D_HID), jnp.float32),
          pltpu.VMEM_SHARED((NP, D_HID), jnp.float32),
      ],
  )
  def deg_kernel(dst_hbm, out_hbm, dst_v, ones_v, zbuf, acc):
    c = lax.axis_index("c")
    s = lax.axis_index("s")
    wid = c * NS + s
    pltpu.sync_copy(dst_hbm.at[wid], dst_v)

    def fill(i, carry):
      for k in range(D_HID // 16):
        ones_v[i, pl.ds(k * 16, 16)] = jnp.ones((16,), jnp.float32)
        zbuf[i, pl.ds(k * 16, 16)] = jnp.zeros((16,), jnp.float32)
      return carry
    lax.fori_loop(0, CH, fill, None)
    _zero_acc_slice(zbuf, acc, s * RPT)
    plsc.subcore_barrier()

    def body(j, carry):
      pltpu.sync_copy(ones_v, acc.at[dst_v.at[j]], add=True)
      return carry
    lax.fori_loop(0, TCH, body, None)
    plsc.subcore_barrier()
    pltpu.sync_copy(acc.at[pl.ds(s * RPT, RPT)], out_hbm.at[c, pl.ds(s * RPT, RPT)])

  return deg_kernel


def _make_scatter_kernel(width):
  @functools.partial(
      pl.kernel,
      out_type=jax.ShapeDtypeStruct((NC, NP, width), jnp.float32),
      mesh=_mesh(),
      scratch_types=[
          pltpu.VMEM((TCH, CH), jnp.int32),
          pltpu.VMEM((TCH, CH), jnp.int32),
          pltpu.VMEM((CH, width), jnp.float32),
          pltpu.VMEM_SHARED((NP, width), jnp.float32),
          pltpu.SemaphoreType.DMA,
      ],
  )
  def scat_kernel(table_hbm, src_hbm, dst_hbm, out_hbm,
                  src_v, dst_v, rows_v, acc, sem):
    c = lax.axis_index("c")
    s = lax.axis_index("s")
    wid = c * NS + s
    pltpu.sync_copy(src_hbm.at[wid], src_v)
    pltpu.sync_copy(dst_hbm.at[wid], dst_v)
    _zero_vmem_rows(rows_v, CH, width)
    _zero_acc_slice(rows_v, acc, s * RPT)
    plsc.subcore_barrier()

    def body(j, carry):
      pltpu.async_copy(table_hbm.at[src_v.at[j]], rows_v, sem).wait()
      pltpu.sync_copy(rows_v, acc.at[dst_v.at[j]], add=True)
      return carry
    lax.fori_loop(0, TCH, body, None)
    plsc.subcore_barrier()
    pltpu.sync_copy(acc.at[pl.ds(s * RPT, RPT)], out_hbm.at[c, pl.ds(s * RPT, RPT)])

  return scat_kernel


_deg = _make_deg_kernel()
_scat_h = _make_scatter_kernel(D_HID)

RB = NP // 8


def _dis_from_partials(degp):
  def body(p_ref, o_ref):
    deg = p_ref[0, :, 0:1] + p_ref[1, :, 0:1] + 1.0
    o_ref[...] = jnp.where(deg > 0, lax.rsqrt(jnp.maximum(deg, 1e-12)), 0.0)
  return pl.pallas_call(
      body,
      grid=(NP // RB,),
      in_specs=[pl.BlockSpec((NC, RB, D_HID), lambda i: (0, i, 0))],
      out_specs=pl.BlockSpec((RB, 1), lambda i: (i, 0)),
      out_shape=jax.ShapeDtypeStruct((NP, 1), jnp.float32))(degp)


def _g1_matmul(xp, W1, dis):
  def body(x_ref, w_ref, d_ref, o_ref):
    o_ref[...] = jnp.dot(x_ref[...], w_ref[...],
                         preferred_element_type=jnp.float32) * d_ref[...]
  return pl.pallas_call(
      body,
      grid=(NP // RB,),
      in_specs=[
          pl.BlockSpec((RB, D_IN), lambda i: (i, 0)),
          pl.BlockSpec((D_IN, D_HID), lambda i: (0, 0)),
          pl.BlockSpec((RB, 1), lambda i: (i, 0)),
      ],
      out_specs=pl.BlockSpec((RB, D_HID), lambda i: (i, 0)),
      out_shape=jax.ShapeDtypeStruct((NP, D_HID), jnp.float32),
  )(xp, W1, dis)


def _u_layer(p1, g1, dis, b1):
  def body(p_ref, g_ref, d_ref, b_ref, o_ref):
    d = d_ref[...]
    agg = p_ref[0] + p_ref[1] + g_ref[...]
    o_ref[...] = jnp.maximum(agg * d + b_ref[...], 0.0) * d
  return pl.pallas_call(
      body,
      grid=(NP // RB,),
      in_specs=[
          pl.BlockSpec((NC, RB, D_HID), lambda i: (0, i, 0)),
          pl.BlockSpec((RB, D_HID), lambda i: (i, 0)),
          pl.BlockSpec((RB, 1), lambda i: (i, 0)),
          pl.BlockSpec((1, D_HID), lambda i: (0, 0)),
      ],
      out_specs=pl.BlockSpec((RB, D_HID), lambda i: (i, 0)),
      out_shape=jax.ShapeDtypeStruct((NP, D_HID), jnp.float32),
  )(p1, g1, dis, b1)


def _final(p2, u, dis, W2, b2):
  def body(p_ref, u_ref, d_ref, w_ref, b_ref, o_ref):
    agg = (p_ref[0] + p_ref[1] + u_ref[...]) * d_ref[...]
    z = jnp.dot(agg, w_ref[...],
                preferred_element_type=jnp.float32) + b_ref[...]
    m = jnp.max(z, axis=1, keepdims=True)
    e = jnp.exp(z - m)
    lse = jnp.log(jnp.sum(e, axis=1, keepdims=True)) + m
    o_ref[...] = z - lse
  return pl.pallas_call(
      body,
      grid=(NP // RB,),
      in_specs=[
          pl.BlockSpec((NC, RB, D_HID), lambda i: (0, i, 0)),
          pl.BlockSpec((RB, D_HID), lambda i: (i, 0)),
          pl.BlockSpec((RB, 1), lambda i: (i, 0)),
          pl.BlockSpec((D_HID, D_OUT), lambda i: (0, 0)),
          pl.BlockSpec((1, D_OUT), lambda i: (0, 0)),
      ],
      out_specs=pl.BlockSpec((RB, D_OUT), lambda i: (i, 0)),
      out_shape=jax.ShapeDtypeStruct((NP, D_OUT), jnp.float32),
  )(p2, u, dis, W2, b2)


def kernel(x, edge_index, W1, b1, W2, b2):
  src = edge_index[0]
  dst = edge_index[1]
  pad = EP - EDGES
  src_flat = jnp.concatenate([src, jnp.zeros((pad,), jnp.int32)])
  dst_flat = jnp.concatenate(
      [dst, NN + (jnp.arange(pad, dtype=jnp.int32) % (NP - NN))])
  src_p = src_flat.reshape(TCH, NC * NS, CH).transpose(1, 0, 2)
  dst_p = dst_flat.reshape(TCH, NC * NS, CH).transpose(1, 0, 2)
  xp = jnp.pad(x, ((0, NP - NN), (0, 0)))

  degp = _deg(dst_p)
  dis = _dis_from_partials(degp)
  g1 = _g1_matmul(xp, W1, dis)
  p1 = _scat_h(g1, src_p, dst_p)
  u = _u_layer(p1, g1, dis, b1.reshape(1, D_HID))
  p2 = _scat_h(u, src_p, dst_p)
  out = _final(p2, u, dis, W2, b2.reshape(1, D_OUT))
  return out[:NN]


# speedup vs baseline: 1.0279x; 1.0279x over previous
import functools

import jax
import jax.numpy as jnp
from jax import lax
from jax.experimental import pallas as pl
from jax.experimental.pallas import tpu as pltpu
from jax.experimental.pallas import tpu_sc as plsc

NN = 10000
NP = 10112
D_IN = 128
D_HID = 128
D_OUT = 64
EDGES = 320000
CH = 128
EP = 323584
NCH = EP // CH
NC = 2
NS = 16
TCH = NCH // (NC * NS)
RPT = NP // NS

_mesh = lambda: plsc.VectorSubcoreMesh(
    core_axis_name="c", subcore_axis_name="s", num_cores=NC, num_subcores=NS)


def _zero_vmem_rows(ref, nrows, width):
  def zrow(i, carry):
    for k in range(width // 16):
      ref[i, pl.ds(k * 16, 16)] = jnp.zeros((16,), jnp.float32)
    return carry
  lax.fori_loop(0, nrows, zrow, None)


def _zero_acc_slice(zbuf, acc, base):
  for t in range(RPT // CH):
    pltpu.sync_copy(zbuf, acc.at[pl.ds(base + t * CH, CH)])
  rem = RPT % CH
  if rem:
    pltpu.sync_copy(zbuf.at[pl.ds(0, rem)], acc.at[pl.ds(base + (RPT // CH) * CH, rem)])


def _make_deg_kernel():
  @functools.partial(
      pl.kernel,
      out_type=jax.ShapeDtypeStruct((NC * NS, NP // 128, 128), jnp.float32),
      mesh=_mesh(),
      compiler_params=pltpu.CompilerParams(needs_layout_passes=False),
      scratch_types=[
          pltpu.VMEM((TCH, CH), jnp.int32),
          pltpu.VMEM((NP // 128, 128), jnp.float32),
      ],
  )
  def deg_kernel(dst_hbm, out_hbm, dst_v, hist):
    c = lax.axis_index("c")
    s = lax.axis_index("s")
    wid = c * NS + s
    pltpu.sync_copy(dst_hbm.at[wid], dst_v)

    def zero(i, carry):
      for k in range(8):
        hist[i, pl.ds(k * 16, 16)] = jnp.zeros((16,), jnp.float32)
      return carry
    lax.fori_loop(0, NP // 128, zero, None)

    def body(j, carry):
      for k in range(CH // 16):
        v = dst_v[j, pl.ds(k * 16, 16)]
        cnt, last = plsc.scan_count(v)
        plsc.addupdate_scatter(
            hist, [v >> 7, v & 127], cnt.astype(jnp.float32), mask=last)
      return carry
    lax.fori_loop(0, TCH, body, None)
    pltpu.sync_copy(hist, out_hbm.at[wid])

  return deg_kernel


def _make_scatter_kernel(width):
  @functools.partial(
      pl.kernel,
      out_type=jax.ShapeDtypeStruct((NC, NP, width), jnp.float32),
      mesh=_mesh(),
      scratch_types=[
          pltpu.VMEM((TCH, CH), jnp.int32),
          pltpu.VMEM((TCH, CH), jnp.int32),
          pltpu.VMEM((CH, width), jnp.float32),
          pltpu.VMEM_SHARED((NP, width), jnp.float32),
          pltpu.SemaphoreType.DMA,
      ],
  )
  def scat_kernel(table_hbm, src_hbm, dst_hbm, out_hbm,
                  src_v, dst_v, rows_v, acc, sem):
    c = lax.axis_index("c")
    s = lax.axis_index("s")
    wid = c * NS + s
    pltpu.sync_copy(src_hbm.at[wid], src_v)
    pltpu.sync_copy(dst_hbm.at[wid], dst_v)
    _zero_vmem_rows(rows_v, CH, width)
    _zero_acc_slice(rows_v, acc, s * RPT)
    plsc.subcore_barrier()

    def body(j, carry):
      pltpu.async_copy(table_hbm.at[src_v.at[j]], rows_v, sem).wait()
      pltpu.sync_copy(rows_v, acc.at[dst_v.at[j]], add=True)
      return carry
    lax.fori_loop(0, TCH, body, None)
    plsc.subcore_barrier()
    pltpu.sync_copy(acc.at[pl.ds(s * RPT, RPT)], out_hbm.at[c, pl.ds(s * RPT, RPT)])

  return scat_kernel


_deg = _make_deg_kernel()
_scat_h = _make_scatter_kernel(D_HID)

RB = NP // 8


def _dis_from_partials(degp3):
  def body(p_ref, o_ref):
    deg = jnp.sum(p_ref[...], axis=0)[0] + 1.0
    dis = jnp.where(deg > 0, lax.rsqrt(jnp.maximum(deg, 1e-12)), 0.0)
    o_ref[...] = jnp.transpose(dis)
  return pl.pallas_call(
      body,
      grid=(NP // 128,),
      in_specs=[pl.BlockSpec((NC * NS, 1, 1, 128), lambda i: (0, i, 0, 0))],
      out_specs=pl.BlockSpec((128, 1), lambda i: (i, 0)),
      out_shape=jax.ShapeDtypeStruct((NP, 1), jnp.float32))(degp3)


def _g1_matmul(xp, W1, dis):
  def body(x_ref, w_ref, d_ref, o_ref):
    o_ref[...] = jnp.dot(x_ref[...], w_ref[...],
                         preferred_element_type=jnp.float32) * d_ref[...]
  return pl.pallas_call(
      body,
      grid=(NP // RB,),
      in_specs=[
          pl.BlockSpec((RB, D_IN), lambda i: (i, 0)),
          pl.BlockSpec((D_IN, D_HID), lambda i: (0, 0)),
          pl.BlockSpec((RB, 1), lambda i: (i, 0)),
      ],
      out_specs=pl.BlockSpec((RB, D_HID), lambda i: (i, 0)),
      out_shape=jax.ShapeDtypeStruct((NP, D_HID), jnp.float32),
  )(xp, W1, dis)


def _u_layer(p1, g1, dis, b1):
  def body(p_ref, g_ref, d_ref, b_ref, o_ref):
    d = d_ref[...]
    agg = p_ref[0] + p_ref[1] + g_ref[...]
    o_ref[...] = jnp.maximum(agg * d + b_ref[...], 0.0) * d
  return pl.pallas_call(
      body,
      grid=(NP // RB,),
      in_specs=[
          pl.BlockSpec((NC, RB, D_HID), lambda i: (0, i, 0)),
          pl.BlockSpec((RB, D_HID), lambda i: (i, 0)),
          pl.BlockSpec((RB, 1), lambda i: (i, 0)),
          pl.BlockSpec((1, D_HID), lambda i: (0, 0)),
      ],
      out_specs=pl.BlockSpec((RB, D_HID), lambda i: (i, 0)),
      out_shape=jax.ShapeDtypeStruct((NP, D_HID), jnp.float32),
  )(p1, g1, dis, b1)


def _final(p2, u, dis, W2, b2):
  def body(p_ref, u_ref, d_ref, w_ref, b_ref, o_ref):
    agg = (p_ref[0] + p_ref[1] + u_ref[...]) * d_ref[...]
    z = jnp.dot(agg, w_ref[...],
                preferred_element_type=jnp.float32) + b_ref[...]
    m = jnp.max(z, axis=1, keepdims=True)
    e = jnp.exp(z - m)
    lse = jnp.log(jnp.sum(e, axis=1, keepdims=True)) + m
    o_ref[...] = z - lse
  return pl.pallas_call(
      body,
      grid=(NP // RB,),
      in_specs=[
          pl.BlockSpec((NC, RB, D_HID), lambda i: (0, i, 0)),
          pl.BlockSpec((RB, D_HID), lambda i: (i, 0)),
          pl.BlockSpec((RB, 1), lambda i: (i, 0)),
          pl.BlockSpec((D_HID, D_OUT), lambda i: (0, 0)),
          pl.BlockSpec((1, D_OUT), lambda i: (0, 0)),
      ],
      out_specs=pl.BlockSpec((RB, D_OUT), lambda i: (i, 0)),
      out_shape=jax.ShapeDtypeStruct((NP, D_OUT), jnp.float32),
  )(p2, u, dis, W2, b2)


def kernel(x, edge_index, W1, b1, W2, b2):
  src = edge_index[0]
  dst = edge_index[1]
  pad = EP - EDGES
  src_flat = jnp.concatenate([src, jnp.zeros((pad,), jnp.int32)])
  dst_flat = jnp.concatenate(
      [dst, NN + (jnp.arange(pad, dtype=jnp.int32) % (NP - NN))])
  src_p = src_flat.reshape(TCH, NC * NS, CH).transpose(1, 0, 2)
  dst_p = dst_flat.reshape(TCH, NC * NS, CH).transpose(1, 0, 2)
  xp = jnp.pad(x, ((0, NP - NN), (0, 0)))

  degp = _deg(dst_p)
  dis = _dis_from_partials(degp.reshape(NC * NS, NP // 128, 1, 128))
  g1 = _g1_matmul(xp, W1, dis)
  p1 = _scat_h(g1, src_p, dst_p)
  u = _u_layer(p1, g1, dis, b1.reshape(1, D_HID))
  p2 = _scat_h(u, src_p, dst_p)
  out = _final(p2, u, dis, W2, b2.reshape(1, D_OUT))
  return out[:NN]
